# baseline (speedup 1.0000x reference)
import functools

import jax
import jax.numpy as jnp
from jax import lax
from jax.experimental import pallas as pl
from jax.experimental.pallas import tpu as pltpu
from jax.experimental.pallas import tpu_sc as plsc


_NC, _NS, _L = 2, 16, 16
_NW = _NC * _NS
_V = 1024
_D = 128
_B = 1024
_ROWS_PER_W = _B // _NW
_VCHUNK = 128
_NCHUNK = _V // _VCHUNK
_NJ = _D // _L


def _sc_pool_kernel(x_hbm, table_hbm, esum_hbm, tch_v, xrow_v, acc_v):
    wid = lax.axis_index("s") * _NC + lax.axis_index("c")
    base = wid * _ROWS_PER_W

    def zero_row(r, _):
        for j in range(_NJ):
            acc_v[r, pl.ds(_L * j, _L)] = jnp.zeros((_L,), jnp.float32)
        return 0

    lax.fori_loop(0, _ROWS_PER_W, zero_row, 0)

    def chunk_body(c, _):
        pltpu.sync_copy(table_hbm.at[pl.ds(c * _VCHUNK, _VCHUNK), :], tch_v)

        def row_body(r, _):
            pltpu.sync_copy(x_hbm.at[pl.ds(base + r, 1), :], xrow_v)
            acc = tuple(acc_v[r, pl.ds(_L * j, _L)] for j in range(_NJ))

            def v_body(g, acc):
                xf = xrow_v[0, pl.ds(c * _VCHUNK + g * _L, _L)].astype(
                    jnp.float32
                )
                acc = list(acc)
                for k in range(_L):
                    xs = xf[k]
                    for j in range(_NJ):
                        acc[j] = acc[j] + xs * tch_v[
                            g * _L + k, pl.ds(_L * j, _L)
                        ]
                return tuple(acc)

            acc = lax.fori_loop(0, _VCHUNK // _L, v_body, acc)
            for j in range(_NJ):
                acc_v[r, pl.ds(_L * j, _L)] = acc[j]
            return 0

        lax.fori_loop(0, _ROWS_PER_W, row_body, 0)
        return 0

    lax.fori_loop(0, _NCHUNK, chunk_body, 0)
    pltpu.sync_copy(acc_v, esum_hbm.at[pl.ds(base, _ROWS_PER_W), :])


def _sc_pool(x, table):
    mesh = plsc.VectorSubcoreMesh(core_axis_name="c", subcore_axis_name="s")
    k = pl.kernel(
        _sc_pool_kernel,
        out_type=jax.ShapeDtypeStruct((_B, _D), jnp.float32),
        mesh=mesh,
        scratch_types=[
            pltpu.VMEM((_VCHUNK, _D), jnp.float32),
            pltpu.VMEM((1, _V), jnp.int32),
            pltpu.VMEM((_ROWS_PER_W, _D), jnp.float32),
        ],
    )
    return k(x, table)


_B_BLK = 512


def _ffn_kernel(x_ref, esum_ref, w1_ref, b1_ref, w2_ref, b2_ref, out_ref):
    cnt = jnp.sum(x_ref[...].astype(jnp.float32), axis=1, keepdims=True)
    e = esum_ref[...] / (cnt + 1e-6)
    h = jnp.maximum(
        jnp.dot(e, w1_ref[...], preferred_element_type=jnp.float32)
        + b1_ref[...],
        0.0,
    )
    out_ref[...] = (
        jnp.sum(h * w2_ref[...], axis=1, keepdims=True) + b2_ref[0, 0]
    )


def kernel(x, table, W1, b1, W2, b2):
    B, V = x.shape
    D = table.shape[1]
    H = W1.shape[1]
    esum = _sc_pool(x, table)
    b1r = b1.reshape(1, H)
    w2r = W2.reshape(1, H)
    b2r = b2.reshape(1, 1)
    grid = (B // _B_BLK,)
    out = pl.pallas_call(
        _ffn_kernel,
        grid=grid,
        in_specs=[
            pl.BlockSpec((_B_BLK, V), lambda i: (i, 0)),
            pl.BlockSpec((_B_BLK, D), lambda i: (i, 0)),
            pl.BlockSpec((D, H), lambda i: (0, 0)),
            pl.BlockSpec((1, H), lambda i: (0, 0)),
            pl.BlockSpec((1, H), lambda i: (0, 0)),
            pl.BlockSpec((1, 1), lambda i: (0, 0)),
        ],
        out_specs=pl.BlockSpec((_B_BLK, 1), lambda i: (i, 0)),
        out_shape=jax.ShapeDtypeStruct((B, 1), jnp.float32),
    )(x, esum, W1, b1r, w2r, b2r)
    return out


# speedup vs baseline: 54.5769x; 54.5769x over previous
import jax
import jax.numpy as jnp
from jax.experimental import pallas as pl
from jax.experimental.pallas import tpu as pltpu


_B_BLK = 512


def _ffn_kernel(x_ref, table_ref, w1_ref, b1_ref, w2_ref, b2_ref, out_ref):
    xi = x_ref[...]
    xb = xi.astype(jnp.bfloat16)
    t = table_ref[...]
    thi = t.astype(jnp.bfloat16)
    tlo = (t - thi.astype(jnp.float32)).astype(jnp.bfloat16)
    s = jnp.dot(xb, thi, preferred_element_type=jnp.float32)
    s += jnp.dot(xb, tlo, preferred_element_type=jnp.float32)
    cnt = jnp.sum(xi, axis=1, keepdims=True).astype(jnp.float32)
    e = s / (cnt + 1e-6)
    h = jnp.maximum(
        jnp.dot(e, w1_ref[...], preferred_element_type=jnp.float32)
        + b1_ref[...],
        0.0,
    )
    out_ref[...] = (
        jnp.sum(h * w2_ref[...], axis=1, keepdims=True) + b2_ref[0, 0]
    )


def kernel(x, table, W1, b1, W2, b2):
    B, V = x.shape
    D = table.shape[1]
    H = W1.shape[1]
    b1r = b1.reshape(1, H)
    w2r = W2.reshape(1, H)
    b2r = b2.reshape(1, 1)
    grid = (B // _B_BLK,)
    out = pl.pallas_call(
        _ffn_kernel,
        grid=grid,
        in_specs=[
            pl.BlockSpec((_B_BLK, V), lambda i: (i, 0)),
            pl.BlockSpec((V, D), lambda i: (0, 0)),
            pl.BlockSpec((D, H), lambda i: (0, 0)),
            pl.BlockSpec((1, H), lambda i: (0, 0)),
            pl.BlockSpec((1, H), lambda i: (0, 0)),
            pl.BlockSpec((1, 1), lambda i: (0, 0)),
        ],
        out_specs=pl.BlockSpec((_B_BLK, 1), lambda i: (i, 0)),
        out_shape=jax.ShapeDtypeStruct((B, 1), jnp.float32),
        compiler_params=pltpu.CompilerParams(
            dimension_semantics=("parallel",),
        ),
    )(x, table, W1, b1r, w2r, b2r)
    return out
